# Initial kernel scaffold; baseline (speedup 1.0000x reference)
import jax
import jax.numpy as jnp
from jax.experimental import pallas as pl


def kernel(features, keys, parent_level_keys):
    raise NotImplementedError("write your pallas kernel here")


# speedup vs baseline: 108.8355x; 108.8355x over previous
import functools

import jax
import jax.numpy as jnp
from jax import lax
from jax.experimental import pallas as pl
from jax.experimental.pallas import tpu as pltpu
from jax.experimental.pallas import tpu_sc as plsc

NEG = -1000000000.0


def _i32(x):
    return jnp.int32(x)

_NC = 2
_NS = 16
_NW = _NC * _NS
_L = 16


@functools.partial(jax.jit, static_argnames=("n", "p", "d"))
def _quad_pool_sc(features, keys32, *, n, p, d):
    N, P, D = n, p, d
    R = 128
    PC = 500
    PPW = P // _NW
    SUBC = PPW // PC
    RPW = N // _NW
    KT = 2000
    NLOG = 19
    NF = D // _L

    assert P % _NW == 0 and PPW % PC == 0
    assert N % _NW == 0 and RPW % KT == 0 and KT % _L == 0
    assert N % R == 0 and D % _L == 0

    mesh = plsc.VectorSubcoreMesh(core_axis_name="c", subcore_axis_name="s")

    @functools.partial(
        pl.kernel,
        out_type=(
            jax.ShapeDtypeStruct((P, D), jnp.float32),
            jax.ShapeDtypeStruct((N,), jnp.int32),
        ),
        mesh=mesh,
        compiler_params=pltpu.CompilerParams(use_tc_tiling_on_sc=False),
        scratch_types=[
            pltpu.VMEM((KT,), jnp.int32),
            pltpu.VMEM((KT,), jnp.int32),
            pltpu.VMEM((_L,), jnp.int32),
            pltpu.VMEM((_L,), jnp.int32),
            pltpu.VMEM((PC, D), jnp.float32),
            pltpu.VMEM((R, D), jnp.float32),
            pltpu.VMEM((R + _L,), jnp.int32),
            pltpu.SemaphoreType.DMA,
        ],
    )
    def k(feat_hbm, keys_hbm, pooled_hbm, pidx_hbm,
          kin, kout, idx_v, gat_v, acc, rows_v, krows_v, sem):
        cid = lax.axis_index("c")
        sid = lax.axis_index("s")
        wid = cid * _NS + sid

        def tile_a(t, c):
            base = wid * RPW + t * KT
            pltpu.sync_copy(keys_hbm.at[pl.ds(base, KT)], kin)

            def shift16(i, c2):
                kout[pl.ds(i * _L, _L)] = kin[pl.ds(i * _L, _L)] >> 2
                return c2

            lax.fori_loop(_i32(0), _i32(KT // _L), shift16, 0)
            pltpu.sync_copy(kout, pidx_hbm.at[pl.ds(base, KT)])
            return c

        lax.fori_loop(_i32(0), _i32(RPW // KT), tile_a, 0)

        lane = lax.iota(jnp.int32, _L)
        j = jnp.minimum(lane, SUBC)
        target = (wid * PPW + j * PC) * 4
        lo0 = jnp.zeros((_L,), jnp.int32)
        hi0 = jnp.full((_L,), N, jnp.int32)

        def bstep(t, carry):
            lo, hi = carry
            mid = (lo + hi) >> 1
            idx_v[...] = jnp.minimum(mid, N - 1)
            pltpu.async_copy(keys_hbm.at[idx_v], gat_v, sem).wait()
            g = gat_v[...]
            live = lo < hi
            lo = jnp.where(jnp.logical_and(live, g < target), mid + 1, lo)
            hi = jnp.where(jnp.logical_and(live, g >= target), mid, hi)
            return lo, hi

        lo, _ = lax.fori_loop(_i32(0), _i32(NLOG), bstep, (lo0, hi0))

        negs = jnp.full((_L,), NEG, jnp.float32)
        zeros = jnp.zeros((_L,), jnp.float32)

        for s in range(SUBC):
            p_base = wid * PPW + s * PC
            r_lo = lo[s]
            r_hi = lo[s + 1]

            def init_row(jr, c2):
                for f in range(NF):
                    acc[jr, pl.ds(f * _L, _L)] = negs
                return c2

            lax.fori_loop(_i32(0), _i32(PC), init_row, 0)

            t0 = r_lo >> 7
            t1 = (r_hi + (R - 1)) >> 7

            def tile(t, c2):
                rb = t * R
                pltpu.sync_copy(feat_hbm.at[pl.ds(rb, R)], rows_v)
                pltpu.sync_copy(keys_hbm.at[pl.ds(rb, R)], krows_v.at[pl.ds(0, R)])
                a = jnp.maximum(r_lo - rb, 0)
                b = jnp.minimum(r_hi - rb, R)

                def row(i, c3):
                    kv = krows_v[pl.ds(i, _L)]
                    lp = (kv[0] >> 2) - p_base
                    for f in range(NF):
                        sl = pl.ds(f * _L, _L)
                        acc[lp, sl] = jnp.maximum(acc[lp, sl], rows_v[i, sl])
                    return c3

                lax.fori_loop(a, b, row, 0)
                return c2

            lax.fori_loop(t0, t1, tile, 0)

            def fin_row(jr, c2):
                for f in range(NF):
                    sl = pl.ds(f * _L, _L)
                    v = acc[jr, sl]
                    acc[jr, sl] = jnp.where(v == negs, zeros, v)
                return c2

            lax.fori_loop(_i32(0), _i32(PC), fin_row, 0)
            pltpu.sync_copy(acc, pooled_hbm.at[pl.ds(p_base, PC)])

    return k(features, keys32)


def kernel(features, keys, parent_level_keys):
    N, D = features.shape
    P = parent_level_keys.shape[0]
    keys32 = keys.astype(jnp.int32)
    pooled, pidx = _quad_pool_sc(features, keys32, n=N, p=P, d=D)
    return (pooled, pidx)


# speedup vs baseline: 132.8973x; 1.2211x over previous
import functools

import jax
import jax.numpy as jnp
from jax import lax
from jax.experimental import pallas as pl
from jax.experimental.pallas import tpu as pltpu
from jax.experimental.pallas import tpu_sc as plsc

NEG = -1000000000.0


def _i32(x):
    return jnp.int32(x)


_NC = 2
_NS = 16
_NW = _NC * _NS
_L = 16


@functools.partial(jax.jit, static_argnames=("n", "p", "d"))
def _quad_pool_sc(features, keys32, *, n, p, d):
    N, P, D = n, p, d
    R = 128
    PC = 500
    PPW = P // _NW
    SUBC = PPW // PC
    RPW = N // _NW
    NLOG = 19
    NF = D // _L

    assert P % _NW == 0 and PPW % PC == 0
    assert N % _NW == 0 and RPW % _L == 0
    assert N % R == 0 and D % _L == 0

    mesh = plsc.VectorSubcoreMesh(core_axis_name="c", subcore_axis_name="s")

    @functools.partial(
        pl.kernel,
        out_type=(
            jax.ShapeDtypeStruct((P, D), jnp.float32),
            jax.ShapeDtypeStruct((N,), jnp.int32),
        ),
        mesh=mesh,
        compiler_params=pltpu.CompilerParams(use_tc_tiling_on_sc=False),
        scratch_types=[
            pltpu.VMEM((RPW,), jnp.int32),
            pltpu.VMEM((RPW,), jnp.int32),
            pltpu.VMEM((_L,), jnp.int32),
            pltpu.VMEM((_L,), jnp.int32),
            pltpu.VMEM((PC, D), jnp.float32),
            pltpu.VMEM((R, D), jnp.float32),
            pltpu.VMEM((R, D), jnp.float32),
            pltpu.VMEM((R + _L,), jnp.int32),
            pltpu.VMEM((R + _L,), jnp.int32),
            pltpu.SemaphoreType.DMA,
            pltpu.SemaphoreType.DMA,
        ],
    )
    def k(feat_hbm, keys_hbm, pooled_hbm, pidx_hbm,
          kin, kout, idx_v, gat_v, acc, rows0, rows1, kr0, kr1, sem0, sem1):
        cid = lax.axis_index("c")
        sid = lax.axis_index("s")
        wid = cid * _NS + sid

        rows_s = (rows0, rows1)
        kr_s = (kr0, kr1)
        sem_s = (sem0, sem1)

        base_a = wid * RPW
        pltpu.sync_copy(keys_hbm.at[pl.ds(base_a, RPW)], kin)

        def shift16(i, c2):
            kout[pl.ds(i * _L, _L)] = kin[pl.ds(i * _L, _L)] >> 2
            return c2

        lax.fori_loop(_i32(0), _i32(RPW // _L), shift16, 0)
        pltpu.sync_copy(kout, pidx_hbm.at[pl.ds(base_a, RPW)])

        lane = lax.iota(jnp.int32, _L)
        j = jnp.minimum(lane, SUBC)
        target = (wid * PPW + j * PC) * 4
        lo0 = jnp.zeros((_L,), jnp.int32)
        hi0 = jnp.full((_L,), N, jnp.int32)

        def bstep(t, carry):
            lo, hi = carry
            mid = (lo + hi) >> 1
            idx_v[...] = jnp.minimum(mid, N - 1)
            pltpu.async_copy(keys_hbm.at[idx_v], gat_v, sem0).wait()
            g = gat_v[...]
            live = lo < hi
            lo = jnp.where(jnp.logical_and(live, g < target), mid + 1, lo)
            hi = jnp.where(jnp.logical_and(live, g >= target), mid, hi)
            return lo, hi

        bnd, _ = lax.fori_loop(_i32(0), _i32(NLOG), bstep, (lo0, hi0))

        negs = jnp.full((_L,), NEG, jnp.float32)
        zeros = jnp.zeros((_L,), jnp.float32)

        def issue(t, slot):
            rb = t * R
            pltpu.make_async_copy(
                feat_hbm.at[pl.ds(rb, R)], rows_s[slot], sem_s[slot]).start()
            pltpu.make_async_copy(
                keys_hbm.at[pl.ds(rb, R)], kr_s[slot].at[pl.ds(0, R)],
                sem_s[slot]).start()

        def wait(slot):
            pltpu.make_async_copy(
                feat_hbm.at[pl.ds(0, R)], rows_s[slot], sem_s[slot]).wait()
            pltpu.make_async_copy(
                keys_hbm.at[pl.ds(0, R)], kr_s[slot].at[pl.ds(0, R)],
                sem_s[slot]).wait()

        for s in range(SUBC):
            p_base = wid * PPW + s * PC
            r_lo = bnd[s]
            r_hi = bnd[s + 1]

            def init_row(jr, c2):
                for f in range(NF):
                    acc[jr, pl.ds(f * _L, _L)] = negs
                return c2

            lax.fori_loop(_i32(0), _i32(PC), init_row, 0)

            t0 = (r_lo >> 7) & ~1
            t1 = (r_hi + (R - 1)) >> 7
            nsteps = (t1 - t0 + 1) >> 1

            def process(t, slot, carry):
                rb = t * R
                a = jnp.maximum(r_lo - rb, 0)
                b = jnp.minimum(r_hi - rb, R)

                def row(i, carry2):
                    kv = kr_s[slot][pl.ds(i, _L)]
                    lp = (kv[0] >> 2) - p_base
                    for f in range(NF):
                        sl = pl.ds(f * _L, _L)
                        acc[lp, sl] = jnp.maximum(acc[lp, sl],
                                                  rows_s[slot][i, sl])
                    return carry2

                return lax.fori_loop(a, b, row, carry)

            carry0 = (_i32(-1), tuple(zeros for _ in range(NF)))

            @pl.when(t0 < t1)
            def _prime():
                issue(t0, 0)

            tcap = jnp.maximum(t1 - 1, t0)

            def step(ii, carry):
                t = t0 + ii * 2
                issue(jnp.minimum(t + 1, tcap), 1)
                wait(0)
                carry = process(t, 0, carry)
                issue(jnp.minimum(t + 2, tcap), 0)
                wait(1)
                return process(t + 1, 1, carry)

            cur_pid, cur = lax.fori_loop(_i32(0), nsteps, step, carry0)

            @pl.when(t0 < t1)
            def _drain():
                wait(0)

            lp_fin = jnp.where(cur_pid < 0, _i32(0), cur_pid - p_base)
            for f in range(NF):
                acc[lp_fin, pl.ds(f * _L, _L)] = cur[f]

            def fin_row(jr, c2):
                for f in range(NF):
                    sl = pl.ds(f * _L, _L)
                    v = acc[jr, sl]
                    acc[jr, sl] = jnp.where(v == negs, zeros, v)
                return c2

            lax.fori_loop(_i32(0), _i32(PC), fin_row, 0)
            pltpu.sync_copy(acc, pooled_hbm.at[pl.ds(p_base, PC)])

    return k(features, keys32)


def kernel(features, keys, parent_level_keys):
    N, D = features.shape
    P = parent_level_keys.shape[0]
    keys32 = keys.astype(jnp.int32)
    pooled, pidx = _quad_pool_sc(features, keys32, n=N, p=P, d=D)
    return (pooled, pidx)


# speedup vs baseline: 132.9622x; 1.0005x over previous
import functools

import jax
import jax.numpy as jnp
from jax import lax
from jax.experimental import pallas as pl
from jax.experimental.pallas import tpu as pltpu
from jax.experimental.pallas import tpu_sc as plsc

NEG = -1000000000.0


def _i32(x):
    return jnp.int32(x)


_NC = 2
_NS = 16
_NW = _NC * _NS
_L = 16


@functools.partial(jax.jit, static_argnames=("n", "p", "d"))
def _quad_pool_sc(features, keys32, *, n, p, d):
    N, P, D = n, p, d
    R = 128
    PC = 500
    PPW = P // _NW
    SUBC = PPW // PC
    RPW = N // _NW
    NLOG = 19
    NF = D // _L

    assert P % _NW == 0 and PPW % PC == 0
    assert N % _NW == 0 and RPW % _L == 0
    assert N % R == 0 and D % _L == 0

    mesh = plsc.VectorSubcoreMesh(core_axis_name="c", subcore_axis_name="s")

    @functools.partial(
        pl.kernel,
        out_type=(
            jax.ShapeDtypeStruct((P, D), jnp.float32),
            jax.ShapeDtypeStruct((N,), jnp.int32),
        ),
        mesh=mesh,
        compiler_params=pltpu.CompilerParams(use_tc_tiling_on_sc=False),
        scratch_types=[
            pltpu.VMEM((RPW,), jnp.int32),
            pltpu.VMEM((RPW,), jnp.int32),
            pltpu.VMEM((_L,), jnp.int32),
            pltpu.VMEM((_L,), jnp.int32),
            pltpu.VMEM((PC, D), jnp.float32),
            pltpu.VMEM((R, D), jnp.float32),
            pltpu.VMEM((R, D), jnp.float32),
            pltpu.VMEM((R + _L,), jnp.int32),
            pltpu.VMEM((R + _L,), jnp.int32),
            pltpu.SemaphoreType.DMA,
            pltpu.SemaphoreType.DMA,
        ],
    )
    def k(feat_hbm, keys_hbm, pooled_hbm, pidx_hbm,
          kin, kout, idx_v, gat_v, acc, rows0, rows1, kr0, kr1, sem0, sem1):
        cid = lax.axis_index("c")
        sid = lax.axis_index("s")
        wid = cid * _NS + sid

        rows_s = (rows0, rows1)
        kr_s = (kr0, kr1)
        sem_s = (sem0, sem1)

        base_a = wid * RPW
        pltpu.sync_copy(keys_hbm.at[pl.ds(base_a, RPW)], kin)

        def shift16(i, c2):
            kout[pl.ds(i * _L, _L)] = kin[pl.ds(i * _L, _L)] >> 2
            return c2

        lax.fori_loop(_i32(0), _i32(RPW // _L), shift16, 0)
        pltpu.sync_copy(kout, pidx_hbm.at[pl.ds(base_a, RPW)])

        lane = lax.iota(jnp.int32, _L)
        j = jnp.minimum(lane, SUBC)
        target = (wid * PPW + j * PC) * 4
        lo0 = jnp.zeros((_L,), jnp.int32)
        hi0 = jnp.full((_L,), N, jnp.int32)

        def bstep(t, carry):
            lo, hi = carry
            mid = (lo + hi) >> 1
            idx_v[...] = jnp.minimum(mid, N - 1)
            pltpu.async_copy(keys_hbm.at[idx_v], gat_v, sem0).wait()
            g = gat_v[...]
            live = lo < hi
            lo = jnp.where(jnp.logical_and(live, g < target), mid + 1, lo)
            hi = jnp.where(jnp.logical_and(live, g >= target), mid, hi)
            return lo, hi

        bnd, _ = lax.fori_loop(_i32(0), _i32(NLOG), bstep, (lo0, hi0))

        negs = jnp.full((_L,), NEG, jnp.float32)
        zeros = jnp.zeros((_L,), jnp.float32)

        def issue(t, slot):
            rb = t * R
            pltpu.make_async_copy(
                feat_hbm.at[pl.ds(rb, R)], rows_s[slot], sem_s[slot]).start()
            pltpu.make_async_copy(
                keys_hbm.at[pl.ds(rb, R)], kr_s[slot].at[pl.ds(0, R)],
                sem_s[slot]).start()

        def wait(slot):
            pltpu.make_async_copy(
                feat_hbm.at[pl.ds(0, R)], rows_s[slot], sem_s[slot]).wait()
            pltpu.make_async_copy(
                keys_hbm.at[pl.ds(0, R)], kr_s[slot].at[pl.ds(0, R)],
                sem_s[slot]).wait()

        for s in range(SUBC):
            p_base = wid * PPW + s * PC
            r_lo = bnd[s]
            r_hi = bnd[s + 1]

            def init_row(jr, c2):
                for f in range(NF):
                    acc[jr, pl.ds(f * _L, _L)] = negs
                return c2

            lax.fori_loop(_i32(0), _i32(PC), init_row, 0)

            t0 = (r_lo >> 7) & ~1
            t1 = (r_hi + (R - 1)) >> 7
            nsteps = (t1 - t0 + 1) >> 1

            def process(t, slot, carry):
                rb = t * R
                a = jnp.maximum(r_lo - rb, 0)
                b = jnp.minimum(r_hi - rb, R)

                def row(i, carry2):
                    kv = kr_s[slot][pl.ds(i, _L)]
                    lp = (kv[0] >> 2) - p_base
                    for f in range(NF):
                        sl = pl.ds(f * _L, _L)
                        acc[lp, sl] = jnp.maximum(acc[lp, sl],
                                                  rows_s[slot][i, sl])
                    return carry2

                return lax.fori_loop(a, b, row, carry)

            carry0 = _i32(0)

            @pl.when(t0 < t1)
            def _prime():
                issue(t0, 0)

            tcap = jnp.maximum(t1 - 1, t0)

            def step(ii, carry):
                t = t0 + ii * 2
                issue(jnp.minimum(t + 1, tcap), 1)
                wait(0)
                carry = process(t, 0, carry)
                issue(jnp.minimum(t + 2, tcap), 0)
                wait(1)
                return process(t + 1, 1, carry)

            lax.fori_loop(_i32(0), nsteps, step, carry0)

            @pl.when(t0 < t1)
            def _drain():
                wait(0)

            def fin_row(jr, c2):
                for f in range(NF):
                    sl = pl.ds(f * _L, _L)
                    v = acc[jr, sl]
                    acc[jr, sl] = jnp.where(v == negs, zeros, v)
                return c2

            lax.fori_loop(_i32(0), _i32(PC), fin_row, 0)
            pltpu.sync_copy(acc, pooled_hbm.at[pl.ds(p_base, PC)])

    return k(features, keys32)


def kernel(features, keys, parent_level_keys):
    N, D = features.shape
    P = parent_level_keys.shape[0]
    keys32 = keys.astype(jnp.int32)
    pooled, pidx = _quad_pool_sc(features, keys32, n=N, p=P, d=D)
    return (pooled, pidx)


# speedup vs baseline: 134.6193x; 1.0125x over previous
import functools

import jax
import jax.numpy as jnp
from jax import lax
from jax.experimental import pallas as pl
from jax.experimental.pallas import tpu as pltpu
from jax.experimental.pallas import tpu_sc as plsc

NEG = -1000000000.0


def _i32(x):
    return jnp.int32(x)


_NC = 2
_NS = 16
_NW = _NC * _NS
_L = 16


@functools.partial(jax.jit, static_argnames=("n", "p", "d"))
def _quad_pool_sc(features, keys32, *, n, p, d):
    N, P, D = n, p, d
    R = 128
    PC = 500
    PPW = P // _NW
    SUBC = PPW // PC
    RPW = N // _NW
    NLOG = 19
    NF = D // _L

    assert P % _NW == 0 and PPW % PC == 0
    assert N % _NW == 0 and RPW % _L == 0
    assert N % R == 0 and D % _L == 0

    mesh = plsc.VectorSubcoreMesh(core_axis_name="c", subcore_axis_name="s")

    @functools.partial(
        pl.kernel,
        out_type=(
            jax.ShapeDtypeStruct((P, D), jnp.float32),
            jax.ShapeDtypeStruct((N,), jnp.int32),
        ),
        mesh=mesh,
        compiler_params=pltpu.CompilerParams(use_tc_tiling_on_sc=False),
        scratch_types=[
            pltpu.VMEM((RPW,), jnp.int32),
            pltpu.VMEM((RPW,), jnp.int32),
            pltpu.VMEM((_L,), jnp.int32),
            pltpu.VMEM((_L,), jnp.int32),
            pltpu.VMEM((PC, D), jnp.float32),
            pltpu.VMEM((R, D), jnp.float32),
            pltpu.VMEM((R, D), jnp.float32),
            pltpu.VMEM((R + _L,), jnp.int32),
            pltpu.VMEM((R + _L,), jnp.int32),
            pltpu.SemaphoreType.DMA,
            pltpu.SemaphoreType.DMA,
        ],
    )
    def k(feat_hbm, keys_hbm, pooled_hbm, pidx_hbm,
          kin, kout, idx_v, gat_v, acc, rows0, rows1, kr0, kr1, sem0, sem1):
        cid = lax.axis_index("c")
        sid = lax.axis_index("s")
        wid = cid * _NS + sid

        rows_s = (rows0, rows1)
        kr_s = (kr0, kr1)
        sem_s = (sem0, sem1)

        base_a = wid * RPW
        pltpu.sync_copy(keys_hbm.at[pl.ds(base_a, RPW)], kin)

        def shift16(i, c2):
            kout[pl.ds(i * _L, _L)] = kin[pl.ds(i * _L, _L)] >> 2
            return c2

        lax.fori_loop(_i32(0), _i32(RPW // _L), shift16, 0)
        pltpu.sync_copy(kout, pidx_hbm.at[pl.ds(base_a, RPW)])

        lane = lax.iota(jnp.int32, _L)
        j = jnp.minimum(lane, SUBC)
        target = (wid * PPW + j * PC) * 4
        lo0 = jnp.zeros((_L,), jnp.int32)
        hi0 = jnp.full((_L,), N, jnp.int32)

        def bstep(t, carry):
            lo, hi = carry
            mid = (lo + hi) >> 1
            idx_v[...] = jnp.minimum(mid, N - 1)
            pltpu.async_copy(keys_hbm.at[idx_v], gat_v, sem0).wait()
            g = gat_v[...]
            live = lo < hi
            lo = jnp.where(jnp.logical_and(live, g < target), mid + 1, lo)
            hi = jnp.where(jnp.logical_and(live, g >= target), mid, hi)
            return lo, hi

        bnd, _ = lax.fori_loop(_i32(0), _i32(NLOG), bstep, (lo0, hi0))

        negs = jnp.full((_L,), NEG, jnp.float32)
        zeros = jnp.zeros((_L,), jnp.float32)

        def issue(t, slot):
            rb = t * R
            pltpu.make_async_copy(
                feat_hbm.at[pl.ds(rb, R)], rows_s[slot], sem_s[slot]).start()
            pltpu.make_async_copy(
                keys_hbm.at[pl.ds(rb, R)], kr_s[slot].at[pl.ds(0, R)],
                sem_s[slot]).start()

        def wait(slot):
            pltpu.make_async_copy(
                feat_hbm.at[pl.ds(0, R)], rows_s[slot], sem_s[slot]).wait()
            pltpu.make_async_copy(
                keys_hbm.at[pl.ds(0, R)], kr_s[slot].at[pl.ds(0, R)],
                sem_s[slot]).wait()

        for s in range(SUBC):
            p_base = wid * PPW + s * PC
            r_lo = bnd[s]
            r_hi = bnd[s + 1]

            def init_row(jr, c2):
                for f in range(NF):
                    acc[jr, pl.ds(f * _L, _L)] = negs
                return c2

            lax.fori_loop(_i32(0), _i32(PC), init_row, 0)

            t0 = (r_lo >> 7) & ~1
            t1 = (r_hi + (R - 1)) >> 7
            nsteps = (t1 - t0 + 1) >> 1

            def process(t, slot, carry):
                rb = t * R
                a = jnp.maximum(r_lo - rb, 0)
                b = jnp.minimum(r_hi - rb, R)

                def row_at(i):
                    kv = kr_s[slot][pl.ds(i, _L)]
                    lp = (kv[0] >> 2) - p_base
                    for f in range(NF):
                        sl = pl.ds(f * _L, _L)
                        acc[lp, sl] = jnp.maximum(acc[lp, sl],
                                                  rows_s[slot][i, sl])

                n4 = jnp.maximum((b - a) >> 2, 0)

                def quad(q, c2):
                    i = a + q * 4
                    for u in range(4):
                        row_at(i + u)
                    return c2

                lax.fori_loop(_i32(0), n4, quad, 0)

                def rem(i, c2):
                    row_at(i)
                    return c2

                lax.fori_loop(a + n4 * 4, b, rem, 0)
                return carry

            carry0 = _i32(0)

            @pl.when(t0 < t1)
            def _prime():
                issue(t0, 0)

            tcap = jnp.maximum(t1 - 1, t0)

            def step(ii, carry):
                t = t0 + ii * 2
                issue(jnp.minimum(t + 1, tcap), 1)
                wait(0)
                carry = process(t, 0, carry)
                issue(jnp.minimum(t + 2, tcap), 0)
                wait(1)
                return process(t + 1, 1, carry)

            lax.fori_loop(_i32(0), nsteps, step, carry0)

            @pl.when(t0 < t1)
            def _drain():
                wait(0)

            def fin_row(jr, c2):
                for f in range(NF):
                    sl = pl.ds(f * _L, _L)
                    v = acc[jr, sl]
                    acc[jr, sl] = jnp.where(v == negs, zeros, v)
                return c2

            lax.fori_loop(_i32(0), _i32(PC), fin_row, 0)
            pltpu.sync_copy(acc, pooled_hbm.at[pl.ds(p_base, PC)])

    return k(features, keys32)


def kernel(features, keys, parent_level_keys):
    N, D = features.shape
    P = parent_level_keys.shape[0]
    keys32 = keys.astype(jnp.int32)
    pooled, pidx = _quad_pool_sc(features, keys32, n=N, p=P, d=D)
    return (pooled, pidx)


# speedup vs baseline: 324.5578x; 2.4109x over previous
import functools

import jax
import jax.numpy as jnp
from jax import lax
from jax.experimental import pallas as pl
from jax.experimental.pallas import tpu as pltpu
from jax.experimental.pallas import tpu_sc as plsc

NEG = -1000000000.0


def _i32(x):
    return jnp.int32(x)


_NC = 2
_NS = 16
_NW = _NC * _NS
_L = 16


@functools.partial(jax.jit, static_argnames=("n", "p", "d"))
def _quad_pool_sc(features, keys32, *, n, p, d):
    N, P, D = n, p, d
    R = 128
    PC = 500
    PPW = P // _NW
    SUBC = PPW // PC
    RPW = N // _NW
    NLOG = 19
    NF = D // _L

    assert P % _NW == 0 and PPW % PC == 0
    assert N % _NW == 0 and RPW % _L == 0
    assert N % R == 0 and D % _L == 0 and R % _L == 0

    mesh = plsc.VectorSubcoreMesh(core_axis_name="c", subcore_axis_name="s")

    @functools.partial(
        pl.kernel,
        out_type=(
            jax.ShapeDtypeStruct((P, D), jnp.float32),
            jax.ShapeDtypeStruct((N,), jnp.int32),
        ),
        mesh=mesh,
        compiler_params=pltpu.CompilerParams(use_tc_tiling_on_sc=False),
        scratch_types=[
            pltpu.VMEM((RPW,), jnp.int32),
            pltpu.VMEM((RPW,), jnp.int32),
            pltpu.VMEM((_L,), jnp.int32),
            pltpu.VMEM((_L,), jnp.int32),
            pltpu.VMEM((PC + 1, D), jnp.float32),
            pltpu.VMEM((R, D), jnp.float32),
            pltpu.VMEM((R, D), jnp.float32),
            pltpu.VMEM((R,), jnp.int32),
            pltpu.VMEM((R,), jnp.int32),
            pltpu.SemaphoreType.DMA,
            pltpu.SemaphoreType.DMA,
        ],
    )
    def k(feat_hbm, keys_hbm, pooled_hbm, pidx_hbm,
          kin, kout, idx_v, gat_v, acc, rows0, rows1, kr0, kr1, sem0, sem1):
        cid = lax.axis_index("c")
        sid = lax.axis_index("s")
        wid = cid * _NS + sid

        rows_s = (rows0, rows1)
        kr_s = (kr0, kr1)
        sem_s = (sem0, sem1)

        base_a = wid * RPW
        pltpu.sync_copy(keys_hbm.at[pl.ds(base_a, RPW)], kin)

        def shift16(i, c2):
            kout[pl.ds(i * _L, _L)] = kin[pl.ds(i * _L, _L)] >> 2
            return c2

        lax.fori_loop(_i32(0), _i32(RPW // _L), shift16, 0)
        pltpu.sync_copy(kout, pidx_hbm.at[pl.ds(base_a, RPW)])

        lane = lax.iota(jnp.int32, _L)
        j = jnp.minimum(lane, SUBC)
        target = (wid * PPW + j * PC) * 4
        lo0 = jnp.zeros((_L,), jnp.int32)
        hi0 = jnp.full((_L,), N, jnp.int32)

        def bstep(t, carry):
            lo, hi = carry
            mid = (lo + hi) >> 1
            idx_v[...] = jnp.minimum(mid, N - 1)
            pltpu.async_copy(keys_hbm.at[idx_v], gat_v, sem0).wait()
            g = gat_v[...]
            live = lo < hi
            lo = jnp.where(jnp.logical_and(live, g < target), mid + 1, lo)
            hi = jnp.where(jnp.logical_and(live, g >= target), mid, hi)
            return lo, hi

        bnd, _ = lax.fori_loop(_i32(0), _i32(NLOG), bstep, (lo0, hi0))

        negs = jnp.full((_L,), NEG, jnp.float32)
        zeros = jnp.zeros((_L,), jnp.float32)

        def issue(t, slot):
            rb = t * R
            pltpu.make_async_copy(
                feat_hbm.at[pl.ds(rb, R)], rows_s[slot], sem_s[slot]).start()
            pltpu.make_async_copy(
                keys_hbm.at[pl.ds(rb, R)], kr_s[slot], sem_s[slot]).start()

        def wait(slot):
            pltpu.make_async_copy(
                feat_hbm.at[pl.ds(0, R)], rows_s[slot], sem_s[slot]).wait()
            pltpu.make_async_copy(
                keys_hbm.at[pl.ds(0, R)], kr_s[slot], sem_s[slot]).wait()

        def sub_chunk(s, c):
            p_base = wid * PPW + s * PC
            sel = bnd.at[jnp.minimum(lane + s, _i32(_L - 1))].get(
                mode="promise_in_bounds")
            r_lo = sel[0]
            r_hi = sel[1]

            def init_row(jr, c2):
                for f in range(NF):
                    acc[jr, pl.ds(f * _L, _L)] = negs
                return c2

            lax.fori_loop(_i32(0), _i32(PC), init_row, 0)

            t0 = (r_lo >> 7) & ~1
            t1 = (r_hi + (R - 1)) >> 7
            nsteps = (t1 - t0 + 1) >> 1

            def process(t, slot):
                rb = t * R
                a = jnp.maximum(r_lo - rb, 0)
                b = jnp.minimum(r_hi - rb, R)

                def group(g, c2):
                    base = g * _L
                    kvec = kr_s[slot][pl.ds(base, _L)]
                    ridx = base + lane
                    valid = jnp.logical_and(ridx >= a, ridx < b)
                    lpv = jnp.where(valid, (kvec >> 2) - p_base, _i32(PC))
                    for u in range(_L):
                        lp = lpv[u]
                        rvec = [rows_s[slot][base + u, pl.ds(f * _L, _L)]
                                for f in range(NF)]
                        for f in range(NF):
                            sl = pl.ds(f * _L, _L)
                            acc[lp, sl] = jnp.maximum(acc[lp, sl], rvec[f])
                    return c2

                lax.fori_loop(_i32(0), _i32(R // _L), group, 0)

            @pl.when(t0 < t1)
            def _prime():
                issue(t0, 0)

            tcap = jnp.maximum(t1 - 1, t0)

            def step(ii, c2):
                t = t0 + ii * 2
                issue(jnp.minimum(t + 1, tcap), 1)
                wait(0)
                process(t, 0)
                issue(jnp.minimum(t + 2, tcap), 0)
                wait(1)
                process(t + 1, 1)
                return c2

            lax.fori_loop(_i32(0), nsteps, step, 0)

            @pl.when(t0 < t1)
            def _drain():
                wait(0)

            def fin_row(jr, c2):
                for f in range(NF):
                    sl = pl.ds(f * _L, _L)
                    v = acc[jr, sl]
                    acc[jr, sl] = jnp.where(v == negs, zeros, v)
                return c2

            lax.fori_loop(_i32(0), _i32(PC), fin_row, 0)
            pltpu.sync_copy(acc.at[pl.ds(0, PC)],
                            pooled_hbm.at[pl.ds(p_base, PC)])
            return c

        lax.fori_loop(_i32(0), _i32(SUBC), sub_chunk, 0)

    return k(features, keys32)


def kernel(features, keys, parent_level_keys):
    N, D = features.shape
    P = parent_level_keys.shape[0]
    keys32 = keys.astype(jnp.int32)
    pooled, pidx = _quad_pool_sc(features, keys32, n=N, p=P, d=D)
    return (pooled, pidx)


# speedup vs baseline: 329.3119x; 1.0146x over previous
import functools

import jax
import jax.numpy as jnp
from jax import lax
from jax.experimental import pallas as pl
from jax.experimental.pallas import tpu as pltpu
from jax.experimental.pallas import tpu_sc as plsc

NEG = -1000000000.0


def _i32(x):
    return jnp.int32(x)


_NC = 2
_NS = 16
_NW = _NC * _NS
_L = 16


@functools.partial(jax.jit, static_argnames=("n", "p", "d"))
def _quad_pool_sc(features, keys32, *, n, p, d):
    N, P, D = n, p, d
    R = 128
    PC = 500
    PPW = P // _NW
    SUBC = PPW // PC
    RPW = N // _NW
    NLOG = 19
    NF = D // _L

    assert P % _NW == 0 and PPW % PC == 0
    assert N % _NW == 0 and RPW % _L == 0
    assert N % R == 0 and D % _L == 0 and R % _L == 0

    mesh = plsc.VectorSubcoreMesh(core_axis_name="c", subcore_axis_name="s")

    @functools.partial(
        pl.kernel,
        out_type=(
            jax.ShapeDtypeStruct((P, D), jnp.float32),
            jax.ShapeDtypeStruct((N,), jnp.int32),
        ),
        mesh=mesh,
        compiler_params=pltpu.CompilerParams(use_tc_tiling_on_sc=False),
        scratch_types=[
            pltpu.VMEM((RPW,), jnp.int32),
            pltpu.VMEM((RPW,), jnp.int32),
            pltpu.VMEM((_L,), jnp.int32),
            pltpu.VMEM((_L,), jnp.int32),
            pltpu.VMEM((PC + 1, D), jnp.float32),
            pltpu.VMEM((R, D), jnp.float32),
            pltpu.VMEM((R, D), jnp.float32),
            pltpu.VMEM((R,), jnp.int32),
            pltpu.VMEM((R,), jnp.int32),
            pltpu.SemaphoreType.DMA,
            pltpu.SemaphoreType.DMA,
        ],
    )
    def k(feat_hbm, keys_hbm, pooled_hbm, pidx_hbm,
          kin, kout, idx_v, gat_v, acc, rows0, rows1, kr0, kr1, sem0, sem1):
        cid = lax.axis_index("c")
        sid = lax.axis_index("s")
        wid = cid * _NS + sid

        rows_s = (rows0, rows1)
        kr_s = (kr0, kr1)
        sem_s = (sem0, sem1)

        base_a = wid * RPW
        pltpu.sync_copy(keys_hbm.at[pl.ds(base_a, RPW)], kin)

        def shift16(i, c2):
            kout[pl.ds(i * _L, _L)] = kin[pl.ds(i * _L, _L)] >> 2
            return c2

        lax.fori_loop(_i32(0), _i32(RPW // _L), shift16, 0)
        pltpu.sync_copy(kout, pidx_hbm.at[pl.ds(base_a, RPW)])

        lane = lax.iota(jnp.int32, _L)
        j = jnp.minimum(lane, SUBC)
        target = (wid * PPW + j * PC) * 4
        lo0 = jnp.zeros((_L,), jnp.int32)
        hi0 = jnp.full((_L,), N, jnp.int32)

        def bstep(t, carry):
            lo, hi = carry
            mid = (lo + hi) >> 1
            idx_v[...] = jnp.minimum(mid, N - 1)
            pltpu.async_copy(keys_hbm.at[idx_v], gat_v, sem0).wait()
            g = gat_v[...]
            live = lo < hi
            lo = jnp.where(jnp.logical_and(live, g < target), mid + 1, lo)
            hi = jnp.where(jnp.logical_and(live, g >= target), mid, hi)
            return lo, hi

        bnd, _ = lax.fori_loop(_i32(0), _i32(NLOG), bstep, (lo0, hi0))

        negs = jnp.full((_L,), NEG, jnp.float32)
        zeros = jnp.zeros((_L,), jnp.float32)

        def issue(t, slot):
            rb = t * R
            pltpu.make_async_copy(
                feat_hbm.at[pl.ds(rb, R)], rows_s[slot], sem_s[slot]).start()
            pltpu.make_async_copy(
                keys_hbm.at[pl.ds(rb, R)], kr_s[slot], sem_s[slot]).start()

        def wait(slot):
            pltpu.make_async_copy(
                feat_hbm.at[pl.ds(0, R)], rows_s[slot], sem_s[slot]).wait()
            pltpu.make_async_copy(
                keys_hbm.at[pl.ds(0, R)], kr_s[slot], sem_s[slot]).wait()

        def sub_chunk(s, c):
            p_base = wid * PPW + s * PC
            sel = bnd.at[jnp.minimum(lane + s, _i32(_L - 1))].get(
                mode="promise_in_bounds")
            r_lo = sel[0]
            r_hi = sel[1]

            def init_row(jr, c2):
                for f in range(NF):
                    acc[jr, pl.ds(f * _L, _L)] = negs
                return c2

            lax.fori_loop(_i32(0), _i32(PC), init_row, 0)

            t0 = (r_lo >> 7) & ~1
            t1 = (r_hi + (R - 1)) >> 7
            nsteps = (t1 - t0 + 1) >> 1

            def process(t, slot):
                rb = t * R
                a = jnp.maximum(r_lo - rb, 0)
                b = jnp.minimum(r_hi - rb, R)

                def group(g, c2):
                    base = g * _L
                    kvec = kr_s[slot][pl.ds(base, _L)]
                    ridx = base + lane
                    valid = jnp.logical_and(ridx >= a, ridx < b)
                    lpv = jnp.where(valid, (kvec >> 2) - p_base, _i32(PC))
                    rvec = [rows_s[slot][base, pl.ds(f * _L, _L)]
                            for f in range(NF)]
                    for u in range(_L):
                        lp = lpv[u]
                        if u + 1 < _L:
                            nxt = [rows_s[slot][base + u + 1,
                                                pl.ds(f * _L, _L)]
                                   for f in range(NF)]
                        for f in range(NF):
                            sl = pl.ds(f * _L, _L)
                            acc[lp, sl] = jnp.maximum(acc[lp, sl], rvec[f])
                        if u + 1 < _L:
                            rvec = nxt
                    return c2

                lax.fori_loop(_i32(0), _i32(R // _L), group, 0)

            @pl.when(t0 < t1)
            def _prime():
                issue(t0, 0)

            tcap = jnp.maximum(t1 - 1, t0)

            def step(ii, c2):
                t = t0 + ii * 2
                issue(jnp.minimum(t + 1, tcap), 1)
                wait(0)
                process(t, 0)
                issue(jnp.minimum(t + 2, tcap), 0)
                wait(1)
                process(t + 1, 1)
                return c2

            lax.fori_loop(_i32(0), nsteps, step, 0)

            @pl.when(t0 < t1)
            def _drain():
                wait(0)

            def fin_row(jr, c2):
                for jj in range(2):
                    for f in range(NF):
                        sl = pl.ds(f * _L, _L)
                        v = acc[jr * 2 + jj, sl]
                        acc[jr * 2 + jj, sl] = jnp.where(v == negs, zeros, v)
                return c2

            lax.fori_loop(_i32(0), _i32(PC // 2), fin_row, 0)
            pltpu.sync_copy(acc.at[pl.ds(0, PC)],
                            pooled_hbm.at[pl.ds(p_base, PC)])
            return c

        lax.fori_loop(_i32(0), _i32(SUBC), sub_chunk, 0)

    return k(features, keys32)


def kernel(features, keys, parent_level_keys):
    N, D = features.shape
    P = parent_level_keys.shape[0]
    keys32 = keys.astype(jnp.int32)
    pooled, pidx = _quad_pool_sc(features, keys32, n=N, p=P, d=D)
    return (pooled, pidx)


# speedup vs baseline: 393.3270x; 1.1944x over previous
import functools

import jax
import jax.numpy as jnp
from jax import lax
from jax.experimental import pallas as pl
from jax.experimental.pallas import tpu as pltpu
from jax.experimental.pallas import tpu_sc as plsc

NEG = -1000000000.0


def _i32(x):
    return jnp.int32(x)


_NC = 2
_NS = 16
_NW = _NC * _NS
_L = 16


@functools.partial(jax.jit, static_argnames=("n", "p", "d"))
def _quad_pool_sc(features, keys32, *, n, p, d):
    N, P, D = n, p, d
    R = 128
    PC = 500
    PPW = P // _NW
    SUBC = PPW // PC
    RPW = N // _NW
    NLOG = 19
    NF = D // _L

    assert P % _NW == 0 and PPW % PC == 0
    assert N % _NW == 0 and RPW % _L == 0
    assert N % R == 0 and D % _L == 0 and R % _L == 0

    mesh = plsc.VectorSubcoreMesh(core_axis_name="c", subcore_axis_name="s")

    @functools.partial(
        pl.kernel,
        out_type=(
            jax.ShapeDtypeStruct((P, D), jnp.float32),
            jax.ShapeDtypeStruct((N,), jnp.int32),
        ),
        mesh=mesh,
        compiler_params=pltpu.CompilerParams(use_tc_tiling_on_sc=False),
        scratch_types=[
            pltpu.VMEM((RPW,), jnp.int32),
            pltpu.VMEM((RPW,), jnp.int32),
            pltpu.VMEM((_L,), jnp.int32),
            pltpu.VMEM((_L,), jnp.int32),
            pltpu.VMEM((PC + 1, D), jnp.float32),
            pltpu.VMEM((R, D), jnp.float32),
            pltpu.VMEM((R, D), jnp.float32),
            pltpu.VMEM((R,), jnp.int32),
            pltpu.VMEM((R,), jnp.int32),
            pltpu.SemaphoreType.DMA,
            pltpu.SemaphoreType.DMA,
        ],
    )
    def k(feat_hbm, keys_hbm, pooled_hbm, pidx_hbm,
          kin, kout, idx_v, gat_v, acc, rows0, rows1, kr0, kr1, sem0, sem1):
        cid = lax.axis_index("c")
        sid = lax.axis_index("s")
        wid = cid * _NS + sid

        rows_s = (rows0, rows1)
        kr_s = (kr0, kr1)
        sem_s = (sem0, sem1)

        base_a = wid * RPW
        pltpu.sync_copy(keys_hbm.at[pl.ds(base_a, RPW)], kin)

        def shift16(i, c2):
            kout[pl.ds(i * _L, _L)] = kin[pl.ds(i * _L, _L)] >> 2
            return c2

        lax.fori_loop(_i32(0), _i32(RPW // _L), shift16, 0)
        pltpu.sync_copy(kout, pidx_hbm.at[pl.ds(base_a, RPW)])

        lane = lax.iota(jnp.int32, _L)
        j = jnp.minimum(lane, SUBC)
        target = (wid * PPW + j * PC) * 4
        lo0 = jnp.zeros((_L,), jnp.int32)
        hi0 = jnp.full((_L,), N, jnp.int32)

        def bstep(t, carry):
            lo, hi = carry
            mid = (lo + hi) >> 1
            idx_v[...] = jnp.minimum(mid, N - 1)
            pltpu.async_copy(keys_hbm.at[idx_v], gat_v, sem0).wait()
            g = gat_v[...]
            live = lo < hi
            lo = jnp.where(jnp.logical_and(live, g < target), mid + 1, lo)
            hi = jnp.where(jnp.logical_and(live, g >= target), mid, hi)
            return lo, hi

        bnd, _ = lax.fori_loop(_i32(0), _i32(NLOG), bstep, (lo0, hi0))

        negs = jnp.full((_L,), NEG, jnp.float32)
        zeros = jnp.zeros((_L,), jnp.float32)

        def issue(t, slot):
            rb = t * R
            pltpu.make_async_copy(
                feat_hbm.at[pl.ds(rb, R)], rows_s[slot], sem_s[slot]).start()
            pltpu.make_async_copy(
                keys_hbm.at[pl.ds(rb, R)], kr_s[slot], sem_s[slot]).start()

        def wait(slot):
            pltpu.make_async_copy(
                feat_hbm.at[pl.ds(0, R)], rows_s[slot], sem_s[slot]).wait()
            pltpu.make_async_copy(
                keys_hbm.at[pl.ds(0, R)], kr_s[slot], sem_s[slot]).wait()

        def sub_chunk(s, c):
            p_base = wid * PPW + s * PC
            sel = bnd.at[jnp.minimum(lane + s, _i32(_L - 1))].get(
                mode="promise_in_bounds")
            r_lo = sel[0]
            r_hi = sel[1]

            def init_row(jr, c2):
                for f in range(NF):
                    acc[jr, pl.ds(f * _L, _L)] = negs
                return c2

            lax.fori_loop(_i32(0), _i32(PC), init_row, 0)

            t0 = (r_lo >> 7) & ~1
            t1 = (r_hi + (R - 1)) >> 7
            nsteps = (t1 - t0 + 1) >> 1

            def process(t, slot):
                rb = t * R
                a = jnp.maximum(r_lo - rb, 0)
                b = jnp.minimum(r_hi - rb, R)

                def group(g, c2):
                    base = g * _L
                    kvec = kr_s[slot][pl.ds(base, _L)]
                    ridx = base + lane
                    valid = jnp.logical_and(ridx >= a, ridx < b)
                    lpv = jnp.where(valid, (kvec >> 2) - p_base, _i32(PC))
                    lps = [lpv[u] for u in range(_L)]
                    rv = [rows_s[slot][base, pl.ds(f * _L, _L)]
                          for f in range(NF)]
                    cur = [jnp.maximum(acc[lps[0], pl.ds(f * _L, _L)], rv[f])
                           for f in range(NF)]
                    for u in range(1, _L):
                        rvu = [rows_s[slot][base + u, pl.ds(f * _L, _L)]
                               for f in range(NF)]
                        same = lps[u] == lps[u - 1]
                        for f in range(NF):
                            acc[lps[u - 1], pl.ds(f * _L, _L)] = cur[f]
                        cur = [jnp.where(same,
                                         jnp.maximum(cur[f], rvu[f]),
                                         jnp.maximum(negs, rvu[f]))
                               for f in range(NF)]
                    for f in range(NF):
                        acc[lps[_L - 1], pl.ds(f * _L, _L)] = cur[f]
                    return c2

                lax.fori_loop(_i32(0), _i32(R // _L), group, 0)

            @pl.when(t0 < t1)
            def _prime():
                issue(t0, 0)

            tcap = jnp.maximum(t1 - 1, t0)

            def step(ii, c2):
                t = t0 + ii * 2
                issue(jnp.minimum(t + 1, tcap), 1)
                wait(0)
                process(t, 0)
                issue(jnp.minimum(t + 2, tcap), 0)
                wait(1)
                process(t + 1, 1)
                return c2

            lax.fori_loop(_i32(0), nsteps, step, 0)

            @pl.when(t0 < t1)
            def _drain():
                wait(0)

            def fin_row(jr, c2):
                for jj in range(2):
                    for f in range(NF):
                        sl = pl.ds(f * _L, _L)
                        v = acc[jr * 2 + jj, sl]
                        acc[jr * 2 + jj, sl] = jnp.where(v == negs, zeros, v)
                return c2

            lax.fori_loop(_i32(0), _i32(PC // 2), fin_row, 0)
            pltpu.sync_copy(acc.at[pl.ds(0, PC)],
                            pooled_hbm.at[pl.ds(p_base, PC)])
            return c

        lax.fori_loop(_i32(0), _i32(SUBC), sub_chunk, 0)

    return k(features, keys32)


def kernel(features, keys, parent_level_keys):
    N, D = features.shape
    P = parent_level_keys.shape[0]
    keys32 = keys.astype(jnp.int32)
    pooled, pidx = _quad_pool_sc(features, keys32, n=N, p=P, d=D)
    return (pooled, pidx)


# speedup vs baseline: 396.8680x; 1.0090x over previous
import functools

import jax
import jax.numpy as jnp
from jax import lax
from jax.experimental import pallas as pl
from jax.experimental.pallas import tpu as pltpu
from jax.experimental.pallas import tpu_sc as plsc

NEG = -1000000000.0


def _i32(x):
    return jnp.int32(x)


_NC = 2
_NS = 16
_NW = _NC * _NS
_L = 16


@functools.partial(jax.jit, static_argnames=("n", "p", "d"))
def _quad_pool_sc(features, keys32, *, n, p, d):
    N, P, D = n, p, d
    R = 128
    PC = 500
    PPW = P // _NW
    SUBC = PPW // PC
    RPW = N // _NW
    NLOG = 19
    NF = D // _L

    assert P % _NW == 0 and PPW % PC == 0
    assert N % _NW == 0 and RPW % _L == 0
    assert N % R == 0 and D % _L == 0 and R % _L == 0

    mesh = plsc.VectorSubcoreMesh(core_axis_name="c", subcore_axis_name="s")

    @functools.partial(
        pl.kernel,
        out_type=(
            jax.ShapeDtypeStruct((P, D), jnp.float32),
            jax.ShapeDtypeStruct((N,), jnp.int32),
        ),
        mesh=mesh,
        compiler_params=pltpu.CompilerParams(use_tc_tiling_on_sc=False),
        scratch_types=[
            pltpu.VMEM((RPW,), jnp.int32),
            pltpu.VMEM((RPW,), jnp.int32),
            pltpu.VMEM((_L,), jnp.int32),
            pltpu.VMEM((_L,), jnp.int32),
            pltpu.VMEM((PC + 1, D), jnp.float32),
            pltpu.VMEM((R, D), jnp.float32),
            pltpu.VMEM((R, D), jnp.float32),
            pltpu.VMEM((R,), jnp.int32),
            pltpu.VMEM((R,), jnp.int32),
            pltpu.SemaphoreType.DMA,
            pltpu.SemaphoreType.DMA,
        ],
    )
    def k(feat_hbm, keys_hbm, pooled_hbm, pidx_hbm,
          kin, kout, idx_v, gat_v, acc, rows0, rows1, kr0, kr1, sem0, sem1):
        cid = lax.axis_index("c")
        sid = lax.axis_index("s")
        wid = cid * _NS + sid

        rows_s = (rows0, rows1)
        kr_s = (kr0, kr1)
        sem_s = (sem0, sem1)

        base_a = wid * RPW
        pltpu.sync_copy(keys_hbm.at[pl.ds(base_a, RPW)], kin)

        def shift16(i, c2):
            kout[pl.ds(i * _L, _L)] = kin[pl.ds(i * _L, _L)] >> 2
            return c2

        lax.fori_loop(_i32(0), _i32(RPW // _L), shift16, 0)
        pltpu.sync_copy(kout, pidx_hbm.at[pl.ds(base_a, RPW)])

        lane = lax.iota(jnp.int32, _L)
        j = jnp.minimum(lane, SUBC)
        target = (wid * PPW + j * PC) * 4
        lo0 = jnp.zeros((_L,), jnp.int32)
        hi0 = jnp.full((_L,), N, jnp.int32)

        def bstep(t, carry):
            lo, hi = carry
            mid = (lo + hi) >> 1
            idx_v[...] = jnp.minimum(mid, N - 1)
            pltpu.async_copy(keys_hbm.at[idx_v], gat_v, sem0).wait()
            g = gat_v[...]
            live = lo < hi
            lo = jnp.where(jnp.logical_and(live, g < target), mid + 1, lo)
            hi = jnp.where(jnp.logical_and(live, g >= target), mid, hi)
            return lo, hi

        bnd, _ = lax.fori_loop(_i32(0), _i32(NLOG), bstep, (lo0, hi0))

        negs = jnp.full((_L,), NEG, jnp.float32)
        zeros = jnp.zeros((_L,), jnp.float32)

        def issue(t, slot):
            rb = t * R
            pltpu.make_async_copy(
                feat_hbm.at[pl.ds(rb, R)], rows_s[slot], sem_s[slot]).start()
            pltpu.make_async_copy(
                keys_hbm.at[pl.ds(rb, R)], kr_s[slot], sem_s[slot]).start()

        def wait(slot):
            pltpu.make_async_copy(
                feat_hbm.at[pl.ds(0, R)], rows_s[slot], sem_s[slot]).wait()
            pltpu.make_async_copy(
                keys_hbm.at[pl.ds(0, R)], kr_s[slot], sem_s[slot]).wait()

        def sub_chunk(s, c):
            p_base = wid * PPW + s * PC
            sel = bnd.at[jnp.minimum(lane + s, _i32(_L - 1))].get(
                mode="promise_in_bounds")
            r_lo = sel[0]
            r_hi = sel[1]

            t0 = (r_lo >> 7) & ~1
            t1 = (r_hi + (R - 1)) >> 7
            nsteps = (t1 - t0 + 1) >> 1
            tcap = jnp.maximum(t1 - 1, t0)

            @pl.when(t0 < t1)
            def _prime():
                issue(t0, 0)
                issue(jnp.minimum(t0 + 1, tcap), 1)

            def init_row(jr, c2):
                for f in range(NF):
                    acc[jr, pl.ds(f * _L, _L)] = negs
                return c2

            lax.fori_loop(_i32(0), _i32(PC), init_row, 0)

            def process(t, slot):
                rb = t * R
                a = jnp.maximum(r_lo - rb, 0)
                b = jnp.minimum(r_hi - rb, R)

                def group(g, c2):
                    base = g * _L
                    kvec = kr_s[slot][pl.ds(base, _L)]
                    ridx = base + lane
                    valid = jnp.logical_and(ridx >= a, ridx < b)
                    lpv = jnp.where(valid, (kvec >> 2) - p_base, _i32(PC))
                    lps = [lpv[u] for u in range(_L)]
                    rv = [rows_s[slot][base, pl.ds(f * _L, _L)]
                          for f in range(NF)]
                    cur = [jnp.maximum(acc[lps[0], pl.ds(f * _L, _L)], rv[f])
                           for f in range(NF)]
                    for u in range(1, _L):
                        rvu = [rows_s[slot][base + u, pl.ds(f * _L, _L)]
                               for f in range(NF)]
                        same = lps[u] == lps[u - 1]
                        for f in range(NF):
                            acc[lps[u - 1], pl.ds(f * _L, _L)] = cur[f]
                        cur = [jnp.where(same,
                                         jnp.maximum(cur[f], rvu[f]),
                                         jnp.maximum(negs, rvu[f]))
                               for f in range(NF)]
                    for f in range(NF):
                        acc[lps[_L - 1], pl.ds(f * _L, _L)] = cur[f]
                    return c2

                lax.fori_loop(_i32(0), _i32(R // _L), group, 0)

            def step(ii, c2):
                t = t0 + ii * 2
                wait(0)
                process(t, 0)
                issue(jnp.minimum(t + 2, tcap), 0)
                wait(1)
                process(t + 1, 1)
                issue(jnp.minimum(t + 3, tcap), 1)
                return c2

            lax.fori_loop(_i32(0), nsteps, step, 0)

            @pl.when(t0 < t1)
            def _drain():
                wait(0)
                wait(1)

            def fin_row(jr, c2):
                for jj in range(2):
                    for f in range(NF):
                        sl = pl.ds(f * _L, _L)
                        v = acc[jr * 2 + jj, sl]
                        acc[jr * 2 + jj, sl] = jnp.where(v == negs, zeros, v)
                return c2

            lax.fori_loop(_i32(0), _i32(PC // 2), fin_row, 0)
            pltpu.sync_copy(acc.at[pl.ds(0, PC)],
                            pooled_hbm.at[pl.ds(p_base, PC)])
            return c

        lax.fori_loop(_i32(0), _i32(SUBC), sub_chunk, 0)

    return k(features, keys32)


def kernel(features, keys, parent_level_keys):
    N, D = features.shape
    P = parent_level_keys.shape[0]
    keys32 = keys.astype(jnp.int32)
    pooled, pidx = _quad_pool_sc(features, keys32, n=N, p=P, d=D)
    return (pooled, pidx)


# speedup vs baseline: 397.4318x; 1.0014x over previous
import functools

import jax
import jax.numpy as jnp
from jax import lax
from jax.experimental import pallas as pl
from jax.experimental.pallas import tpu as pltpu
from jax.experimental.pallas import tpu_sc as plsc

NEG = -1000000000.0


def _i32(x):
    return jnp.int32(x)


_NC = 2
_NS = 16
_NW = _NC * _NS
_L = 16


@functools.partial(jax.jit, static_argnames=("n", "p", "d"))
def _quad_pool_sc(features, keys32, *, n, p, d):
    N, P, D = n, p, d
    R = 128
    PC = 500
    PPW = P // _NW
    SUBC = PPW // PC
    RPW = N // _NW
    NLOG = 19
    NF = D // _L

    assert P % _NW == 0 and PPW % PC == 0
    assert N % _NW == 0 and RPW % _L == 0
    assert N % R == 0 and D % _L == 0 and R % _L == 0

    mesh = plsc.VectorSubcoreMesh(core_axis_name="c", subcore_axis_name="s")

    @functools.partial(
        pl.kernel,
        out_type=(
            jax.ShapeDtypeStruct((P, D), jnp.float32),
            jax.ShapeDtypeStruct((N,), jnp.int32),
        ),
        mesh=mesh,
        compiler_params=pltpu.CompilerParams(use_tc_tiling_on_sc=False),
        scratch_types=[
            pltpu.VMEM((RPW,), jnp.int32),
            pltpu.VMEM((RPW,), jnp.int32),
            pltpu.VMEM((_L,), jnp.int32),
            pltpu.VMEM((_L,), jnp.int32),
            pltpu.VMEM((PC + 1, D), jnp.float32),
            pltpu.VMEM((R, D), jnp.float32),
            pltpu.VMEM((R, D), jnp.float32),
            pltpu.VMEM((R,), jnp.int32),
            pltpu.VMEM((R,), jnp.int32),
            pltpu.SemaphoreType.DMA,
            pltpu.SemaphoreType.DMA,
        ],
    )
    def k(feat_hbm, keys_hbm, pooled_hbm, pidx_hbm,
          kin, kout, idx_v, gat_v, acc, rows0, rows1, kr0, kr1, sem0, sem1):
        cid = lax.axis_index("c")
        sid = lax.axis_index("s")
        wid = cid * _NS + sid

        rows_s = (rows0, rows1)
        kr_s = (kr0, kr1)
        sem_s = (sem0, sem1)

        base_a = wid * RPW
        pltpu.sync_copy(keys_hbm.at[pl.ds(base_a, RPW)], kin)

        def shift16(i, c2):
            kout[pl.ds(i * _L, _L)] = kin[pl.ds(i * _L, _L)] >> 2
            return c2

        lax.fori_loop(_i32(0), _i32(RPW // _L), shift16, 0)
        pltpu.sync_copy(kout, pidx_hbm.at[pl.ds(base_a, RPW)])

        lane = lax.iota(jnp.int32, _L)
        j = jnp.minimum(lane, SUBC)
        target = (wid * PPW + j * PC) * 4
        lo0 = jnp.zeros((_L,), jnp.int32)
        hi0 = jnp.full((_L,), N, jnp.int32)

        def bstep(t, carry):
            lo, hi = carry
            mid = (lo + hi) >> 1
            idx_v[...] = jnp.minimum(mid, N - 1)
            pltpu.async_copy(keys_hbm.at[idx_v], gat_v, sem0).wait()
            g = gat_v[...]
            live = lo < hi
            lo = jnp.where(jnp.logical_and(live, g < target), mid + 1, lo)
            hi = jnp.where(jnp.logical_and(live, g >= target), mid, hi)
            return lo, hi

        bnd, _ = lax.fori_loop(_i32(0), _i32(NLOG), bstep, (lo0, hi0))

        negs = jnp.full((_L,), NEG, jnp.float32)
        zeros = jnp.zeros((_L,), jnp.float32)

        def issue(t, slot):
            rb = t * R
            pltpu.make_async_copy(
                feat_hbm.at[pl.ds(rb, R)], rows_s[slot], sem_s[slot]).start()
            pltpu.make_async_copy(
                keys_hbm.at[pl.ds(rb, R)], kr_s[slot], sem_s[slot]).start()

        def wait(slot):
            pltpu.make_async_copy(
                feat_hbm.at[pl.ds(0, R)], rows_s[slot], sem_s[slot]).wait()
            pltpu.make_async_copy(
                keys_hbm.at[pl.ds(0, R)], kr_s[slot], sem_s[slot]).wait()

        def sub_chunk(s, c):
            p_base = wid * PPW + s * PC
            sel = bnd.at[jnp.minimum(lane + s, _i32(_L - 1))].get(
                mode="promise_in_bounds")
            r_lo = sel[0]
            r_hi = sel[1]

            t0 = (r_lo >> 7) & ~1
            t1 = (r_hi + (R - 1)) >> 7
            nsteps = (t1 - t0 + 1) >> 1
            tcap = jnp.maximum(t1 - 1, t0)

            @pl.when(t0 < t1)
            def _prime():
                issue(t0, 0)
                issue(jnp.minimum(t0 + 1, tcap), 1)

            def init_row(jr, c2):
                for f in range(NF):
                    acc[jr, pl.ds(f * _L, _L)] = negs
                return c2

            lax.fori_loop(_i32(0), _i32(PC), init_row, 0)

            def process(t, slot):
                rb = t * R
                a = jnp.maximum(r_lo - rb, 0)
                b = jnp.minimum(r_hi - rb, R)

                def group(g, c2):
                    base = g * _L
                    kvec = kr_s[slot][pl.ds(base, _L)]
                    ridx = base + lane
                    valid = jnp.logical_and(ridx >= a, ridx < b)
                    lpv = jnp.where(valid, (kvec >> 2) - p_base, _i32(PC))
                    lps = [lpv[u] for u in range(_L)]
                    rv = [rows_s[slot][base, pl.ds(f * _L, _L)]
                          for f in range(NF)]
                    cur = [jnp.maximum(acc[lps[0], pl.ds(f * _L, _L)], rv[f])
                           for f in range(NF)]
                    for u in range(1, _L):
                        rvu = [rows_s[slot][base + u, pl.ds(f * _L, _L)]
                               for f in range(NF)]
                        same = lps[u] == lps[u - 1]
                        for f in range(NF):
                            acc[lps[u - 1], pl.ds(f * _L, _L)] = cur[f]
                        cur = [jnp.maximum(jnp.where(same, cur[f], negs),
                                           rvu[f])
                               for f in range(NF)]
                    for f in range(NF):
                        acc[lps[_L - 1], pl.ds(f * _L, _L)] = cur[f]
                    return c2

                lax.fori_loop(_i32(0), _i32(R // _L), group, 0)

            def step(ii, c2):
                t = t0 + ii * 2
                wait(0)
                process(t, 0)
                issue(jnp.minimum(t + 2, tcap), 0)
                wait(1)
                process(t + 1, 1)
                issue(jnp.minimum(t + 3, tcap), 1)
                return c2

            lax.fori_loop(_i32(0), nsteps, step, 0)

            @pl.when(t0 < t1)
            def _drain():
                wait(0)
                wait(1)

            def fin_row(jr, c2):
                for jj in range(2):
                    for f in range(NF):
                        sl = pl.ds(f * _L, _L)
                        v = acc[jr * 2 + jj, sl]
                        acc[jr * 2 + jj, sl] = jnp.where(v == negs, zeros, v)
                return c2

            lax.fori_loop(_i32(0), _i32(PC // 2), fin_row, 0)
            pltpu.sync_copy(acc.at[pl.ds(0, PC)],
                            pooled_hbm.at[pl.ds(p_base, PC)])
            return c

        lax.fori_loop(_i32(0), _i32(SUBC), sub_chunk, 0)

    return k(features, keys32)


def kernel(features, keys, parent_level_keys):
    N, D = features.shape
    P = parent_level_keys.shape[0]
    keys32 = keys.astype(jnp.int32)
    pooled, pidx = _quad_pool_sc(features, keys32, n=N, p=P, d=D)
    return (pooled, pidx)
